# baseline (speedup 1.0000x reference)
import functools

import jax
import jax.numpy as jnp
from jax import lax
from jax.experimental import pallas as pl
from jax.experimental.pallas import tpu as pltpu
from jax.experimental.pallas import tpu_sc as plsc

N = 10000
E = 320000
D = 128
H = 128
G = 64
C = 10

NC = 2
NS = 16
NW = NC * NS
CHUNK = 80
ROWS_TOT = E // CHUNK
ROWS_W = ROWS_TOT // NW
EDGES_W = E // NW
N2 = 10240
TILE_ROWS = N2 // NS
ZROWS = 80


def _minadj_body(ei_ref, out_ref):
    ei = ei_ref[...]
    out_ref[...] = ei - jnp.min(ei)


def _min_adjust(edge_index):
    return pl.pallas_call(
        _minadj_body,
        out_shape=jax.ShapeDtypeStruct((2, E), jnp.int32),
    )(edge_index)


def _fill16(ref, nrows, ncols, val):
    per_row = ncols // 16

    def body(i, _):
        r = i // per_row
        c = (i % per_row) * 16
        ref[r, pl.ds(c, 16)] = jnp.full((16,), val, jnp.float32)
        return 0

    lax.fori_loop(0, nrows * per_row, body, 0)


def _sc_agg_body(h_hbm, src_hbm, dst_hbm, agg_out, srcv, dstv, rows, agg_sp, sem, ssem):
    cid = lax.axis_index("c")
    sid = lax.axis_index("s")
    wid = sid * NC + cid

    _fill16(rows.at[0], ZROWS, 128, 0.0)
    for k in range(TILE_ROWS // ZROWS):
        pltpu.sync_copy(rows.at[0],
                        agg_sp.at[pl.ds(sid * TILE_ROWS + k * ZROWS, ZROWS)])
    plsc.subcore_barrier()

    pltpu.sync_copy(src_hbm.at[wid], srcv)
    pltpu.sync_copy(dst_hbm.at[wid], dstv)

    def gather_idx(j):
        return srcv.at[pl.ds(j * CHUNK, CHUNK)]

    pltpu.async_copy(h_hbm.at[gather_idx(0)], rows.at[0], sem)

    def scat_wait(b):
        pltpu.make_async_copy(rows.at[b], agg_sp.at[pl.ds(0, ZROWS)],
                              ssem).wait()

    def chunk_body(j, _):
        b = lax.rem(j, 2)
        pltpu.make_async_copy(h_hbm.at[gather_idx(j)], rows.at[b], sem).wait()

        @pl.when(j >= 1)
        def _():
            scat_wait(1 - b)

        @pl.when(j + 1 < ROWS_W)
        def _():
            pltpu.async_copy(h_hbm.at[gather_idx(j + 1)], rows.at[1 - b], sem)

        pltpu.async_copy(rows.at[b], agg_sp.at[dstv.at[j]], ssem, add=True)
        return 0

    lax.fori_loop(0, ROWS_W, chunk_body, 0)
    scat_wait(lax.rem(ROWS_W - 1, 2))
    plsc.subcore_barrier()

    for k in range(TILE_ROWS // ZROWS):
        r = sid * TILE_ROWS + k * ZROWS
        pltpu.sync_copy(agg_sp.at[pl.ds(r, ZROWS)], rows.at[0])
        pltpu.sync_copy(rows.at[0], agg_out.at[cid, pl.ds(r, ZROWS)])


def _sc_deg_body(dst_hbm, deg_out, dstv, degloc, tmpa, tmpb, parts_sp, sem):
    cid = lax.axis_index("c")
    sid = lax.axis_index("s")
    wid = sid * NC + cid

    def zero16(i, _):
        degloc[pl.ds(i * 16, 16)] = jnp.zeros((16,), jnp.float32)
        return 0

    lax.fori_loop(0, N2 // 16, zero16, 0)
    pltpu.sync_copy(dst_hbm.at[wid], dstv)

    def hist_row(j, _):
        for c in range(CHUNK // 16):
            idx = dstv[j, pl.ds(c * 16, 16)]
            cnt, last = plsc.scan_count(idx)
            plsc.addupdate_scatter(degloc, [idx], cnt.astype(jnp.float32),
                                   mask=last)
        return 0

    lax.fori_loop(0, ROWS_W, hist_row, 0)

    pltpu.sync_copy(degloc, parts_sp.at[sid])
    plsc.subcore_barrier()
    base = sid * TILE_ROWS
    pltpu.sync_copy(parts_sp.at[0, pl.ds(base, TILE_ROWS)], tmpa)
    for r in range(1, NS):
        pltpu.sync_copy(parts_sp.at[r, pl.ds(base, TILE_ROWS)], tmpb)
        for c in range(TILE_ROWS // 16):
            s = pl.ds(c * 16, 16)
            tmpa[s] = tmpa[s] + tmpb[s]
    pltpu.sync_copy(tmpa, deg_out.at[cid, pl.ds(base, TILE_ROWS)])


@functools.lru_cache(maxsize=None)
def _sc_pass():
    mesh = plsc.VectorSubcoreMesh(core_axis_name="c", subcore_axis_name="s")
    return pl.kernel(
        _sc_agg_body,
        out_type=jax.ShapeDtypeStruct((NC, N2, 128), jnp.float32),
        mesh=mesh,
        scratch_types=[
            pltpu.VMEM((EDGES_W,), jnp.int32),
            pltpu.VMEM((ROWS_W, CHUNK), jnp.int32),
            pltpu.VMEM((2, ZROWS, 128), jnp.float32),
            pltpu.VMEM_SHARED((N2, 128), jnp.float32),
            pltpu.SemaphoreType.DMA,
            pltpu.SemaphoreType.DMA,
        ],
    )


@functools.lru_cache(maxsize=None)
def _sc_deg():
    mesh = plsc.VectorSubcoreMesh(core_axis_name="c", subcore_axis_name="s")
    return pl.kernel(
        _sc_deg_body,
        out_type=jax.ShapeDtypeStruct((NC, N2), jnp.float32),
        mesh=mesh,
        compiler_params=pltpu.CompilerParams(needs_layout_passes=False),
        scratch_types=[
            pltpu.VMEM((ROWS_W, CHUNK), jnp.int32),
            pltpu.VMEM((N2,), jnp.float32),
            pltpu.VMEM((TILE_ROWS,), jnp.float32),
            pltpu.VMEM((TILE_ROWS,), jnp.float32),
            pltpu.VMEM_SHARED((NS, N2), jnp.float32),
            pltpu.SemaphoreType.DMA,
        ],
    )


_BM = 1000


def _layer_body(h_ref, a0_ref, a1_ref, d0_ref, d1_ref, ws_ref, wn_ref, b_ref,
                out_ref):
    deg = d0_ref[...] + d1_ref[...]
    mean = (a0_ref[...] + a1_ref[...]) / jnp.maximum(deg, 1.0)
    acc = jnp.dot(h_ref[...], ws_ref[...], preferred_element_type=jnp.float32)
    acc = acc + jnp.dot(mean, wn_ref[...], preferred_element_type=jnp.float32)
    out_ref[...] = jnp.maximum(acc + b_ref[...], 0.0)


def _layer(h, a0, a1, d0, d1, ws, wn, b):
    grid = N // _BM
    return pl.pallas_call(
        _layer_body,
        grid=(grid,),
        in_specs=[
            pl.BlockSpec((_BM, 128), lambda i: (i, 0)),
            pl.BlockSpec((_BM, 128), lambda i: (i, 0)),
            pl.BlockSpec((_BM, 128), lambda i: (i, 0)),
            pl.BlockSpec((_BM, 1), lambda i: (i, 0)),
            pl.BlockSpec((_BM, 1), lambda i: (i, 0)),
            pl.BlockSpec((128, 128), lambda i: (0, 0)),
            pl.BlockSpec((128, 128), lambda i: (0, 0)),
            pl.BlockSpec((1, 128), lambda i: (0, 0)),
        ],
        out_specs=pl.BlockSpec((_BM, 128), lambda i: (i, 0)),
        out_shape=jax.ShapeDtypeStruct((N, 128), jnp.float32),
    )(h, a0, a1, d0, d1, ws, wn, b.reshape(1, 128))


def _pool_body(b_ref, h1_ref, h2_ref, h3_ref, w1a_ref, w1b_ref, w1c_ref,
               fb1_ref, w2_ref, fb2_ref, out_ref, s1, s2, s3, cnt):
    i = pl.program_id(0)

    @pl.when(i == 0)
    def _():
        s1[...] = jnp.zeros_like(s1)
        s2[...] = jnp.zeros_like(s2)
        s3[...] = jnp.zeros_like(s3)
        cnt[...] = jnp.zeros_like(cnt)

    oh = (b_ref[...] == lax.broadcasted_iota(jnp.int32, (1, G), 1)
          ).astype(jnp.float32)
    dn = (((0,), (0,)), ((), ()))
    s1[...] += lax.dot_general(oh, h1_ref[...], dn,
                               preferred_element_type=jnp.float32)
    s2[...] += lax.dot_general(oh, h2_ref[...], dn,
                               preferred_element_type=jnp.float32)
    s3[...] += lax.dot_general(oh, h3_ref[...], dn,
                               preferred_element_type=jnp.float32)
    cnt[...] += lax.dot_general(oh, jnp.ones_like(h1_ref[...]), dn,
                                preferred_element_type=jnp.float32)

    @pl.when(i == pl.num_programs(0) - 1)
    def _():
        inv = 1.0 / jnp.maximum(cnt[...], 1.0)
        t = jnp.dot(s1[...] * inv, w1a_ref[...],
                    preferred_element_type=jnp.float32)
        t = t + jnp.dot(s2[...] * inv, w1b_ref[...],
                        preferred_element_type=jnp.float32)
        t = t + jnp.dot(s3[...] * inv, w1c_ref[...],
                        preferred_element_type=jnp.float32)
        t = jnp.maximum(t + fb1_ref[...], 0.0)
        out_ref[...] = jnp.dot(t, w2_ref[...],
                               preferred_element_type=jnp.float32) + fb2_ref[...]


def _pool_mlp(batch, h1, h2, h3, fc1_W, fc1_b, fc2_W, fc2_b):
    grid = N // _BM
    w1a, w1b, w1c = fc1_W[:128], fc1_W[128:256], fc1_W[256:]
    return pl.pallas_call(
        _pool_body,
        grid=(grid,),
        in_specs=[
            pl.BlockSpec((_BM, 1), lambda i: (i, 0)),
            pl.BlockSpec((_BM, 128), lambda i: (i, 0)),
            pl.BlockSpec((_BM, 128), lambda i: (i, 0)),
            pl.BlockSpec((_BM, 128), lambda i: (i, 0)),
            pl.BlockSpec((128, 128), lambda i: (0, 0)),
            pl.BlockSpec((128, 128), lambda i: (0, 0)),
            pl.BlockSpec((128, 128), lambda i: (0, 0)),
            pl.BlockSpec((1, 128), lambda i: (0, 0)),
            pl.BlockSpec((128, C), lambda i: (0, 0)),
            pl.BlockSpec((1, C), lambda i: (0, 0)),
        ],
        out_specs=pl.BlockSpec((G, C), lambda i: (0, 0)),
        out_shape=jax.ShapeDtypeStruct((G, C), jnp.float32),
        scratch_shapes=[
            pltpu.VMEM((G, 128), jnp.float32),
            pltpu.VMEM((G, 128), jnp.float32),
            pltpu.VMEM((G, 128), jnp.float32),
            pltpu.VMEM((G, 128), jnp.float32),
        ],
    )(batch.reshape(N, 1), h1, h2, h3, w1a, w1b, w1c,
      fc1_b.reshape(1, 128), fc2_W, fc2_b.reshape(1, C))


def kernel(x, edge_index, batch, W_self_0, W_neigh_0, b_0, W_self_1, W_neigh_1,
           b_1, W_self_2, W_neigh_2, b_2, fc1_W, fc1_b, fc2_W, fc2_b):
    ei = _min_adjust(edge_index)
    src = ei[0].reshape(NW, EDGES_W)
    dst = ei[1].reshape(NW, ROWS_W, CHUNK)

    deg = _sc_deg()(dst)
    d0 = deg[0, :N].reshape(N, 1)
    d1 = deg[1, :N].reshape(N, 1)
    agg = _sc_pass()(x, src, dst)
    h1 = _layer(x, agg[0, :N], agg[1, :N], d0, d1, W_self_0, W_neigh_0, b_0)
    agg = _sc_pass()(h1, src, dst)
    h2 = _layer(h1, agg[0, :N], agg[1, :N], d0, d1, W_self_1, W_neigh_1, b_1)
    agg = _sc_pass()(h2, src, dst)
    h3 = _layer(h2, agg[0, :N], agg[1, :N], d0, d1, W_self_2, W_neigh_2, b_2)

    return _pool_mlp(batch, h1, h2, h3, fc1_W, fc1_b, fc2_W, fc2_b)


# speedup vs baseline: 1.3872x; 1.3872x over previous
import functools

import jax
import jax.numpy as jnp
from jax import lax
from jax.experimental import pallas as pl
from jax.experimental.pallas import tpu as pltpu
from jax.experimental.pallas import tpu_sc as plsc

N = 10000
E = 320000
D = 128
H = 128
G = 64
C = 10

NC = 2
NS = 16
NW = NC * NS
CHUNK = 80
ROWS_TOT = E // CHUNK
ROWS_W = ROWS_TOT // NW
EDGES_W = E // NW
N2 = 10240
TILE_ROWS = N2 // NS
ZROWS = 80
DHALF = 64


def _minadj_body(ei_ref, out_ref):
    ei = ei_ref[...]
    out_ref[...] = ei - jnp.min(ei)


def _min_adjust(edge_index):
    return pl.pallas_call(
        _minadj_body,
        out_shape=jax.ShapeDtypeStruct((2, E), jnp.int32),
    )(edge_index)


def _fill16(ref, nrows, ncols, val):
    per_row = ncols // 16

    def body(i, _):
        r = i // per_row
        c = (i % per_row) * 16
        ref[r, pl.ds(c, 16)] = jnp.full((16,), val, jnp.float32)
        return 0

    lax.fori_loop(0, nrows * per_row, body, 0)


def _sc_agg_body(h_hbm, src_hbm, dst_hbm, agg_out, srcv, dstv, rows, agg_sp, sem, ssem):
    cid = lax.axis_index("c")
    sid = lax.axis_index("s")
    wid = sid * NC + cid

    _fill16(rows.at[0], ZROWS, 128, 0.0)
    for k in range(TILE_ROWS // ZROWS):
        pltpu.sync_copy(rows.at[0],
                        agg_sp.at[pl.ds(sid * TILE_ROWS + k * ZROWS, ZROWS)])
    plsc.subcore_barrier()

    pltpu.sync_copy(src_hbm.at[wid], srcv)
    pltpu.sync_copy(dst_hbm.at[wid, pl.ds(0, DHALF)], dstv)

    def gather_idx(j):
        return srcv.at[pl.ds(j * CHUNK, CHUNK)]

    def scat_wait():
        pltpu.make_async_copy(rows.at[0], agg_sp.at[pl.ds(0, ZROWS)],
                              ssem).wait()

    pltpu.async_copy(h_hbm.at[gather_idx(0)], rows.at[0], sem)
    pltpu.async_copy(h_hbm.at[gather_idx(1)], rows.at[1], sem)

    def chunk_body(j, _):
        b = lax.rem(j, 3)
        pltpu.make_async_copy(h_hbm.at[gather_idx(j)], rows.at[b], sem).wait()

        @pl.when(j >= 1)
        def _():
            scat_wait()

        @pl.when(j == DHALF)
        def _():
            pltpu.sync_copy(dst_hbm.at[wid, pl.ds(DHALF, ROWS_W - DHALF)],
                            dstv.at[pl.ds(0, ROWS_W - DHALF)])

        @pl.when(j + 2 < ROWS_W)
        def _():
            pltpu.async_copy(h_hbm.at[gather_idx(j + 2)],
                             rows.at[lax.rem(j + 2, 3)], sem)

        pltpu.async_copy(rows.at[b], agg_sp.at[dstv.at[lax.rem(j, DHALF)]],
                         ssem, add=True)
        return 0

    lax.fori_loop(0, ROWS_W, chunk_body, 0)
    scat_wait()
    plsc.subcore_barrier()

    for k in range(TILE_ROWS // ZROWS):
        r = sid * TILE_ROWS + k * ZROWS
        pltpu.sync_copy(agg_sp.at[pl.ds(r, ZROWS)], rows.at[0])
        pltpu.sync_copy(rows.at[0], agg_out.at[cid, pl.ds(r, ZROWS)])


def _sc_deg_body(dst_hbm, deg_out, dstv, degloc, tmpa, tmpb, parts_sp, sem):
    cid = lax.axis_index("c")
    sid = lax.axis_index("s")
    wid = sid * NC + cid

    def zero16(i, _):
        degloc[pl.ds(i * 16, 16)] = jnp.zeros((16,), jnp.float32)
        return 0

    lax.fori_loop(0, N2 // 16, zero16, 0)
    pltpu.sync_copy(dst_hbm.at[wid], dstv)

    def hist_row(j, _):
        for c in range(CHUNK // 16):
            idx = dstv[j, pl.ds(c * 16, 16)]
            cnt, last = plsc.scan_count(idx)
            plsc.addupdate_scatter(degloc, [idx], cnt.astype(jnp.float32),
                                   mask=last)
        return 0

    lax.fori_loop(0, ROWS_W, hist_row, 0)

    pltpu.sync_copy(degloc, parts_sp.at[sid])
    plsc.subcore_barrier()
    base = sid * TILE_ROWS
    pltpu.sync_copy(parts_sp.at[0, pl.ds(base, TILE_ROWS)], tmpa)
    for r in range(1, NS):
        pltpu.sync_copy(parts_sp.at[r, pl.ds(base, TILE_ROWS)], tmpb)
        for c in range(TILE_ROWS // 16):
            s = pl.ds(c * 16, 16)
            tmpa[s] = tmpa[s] + tmpb[s]
    pltpu.sync_copy(tmpa, deg_out.at[cid, pl.ds(base, TILE_ROWS)])


@functools.lru_cache(maxsize=None)
def _sc_pass():
    mesh = plsc.VectorSubcoreMesh(core_axis_name="c", subcore_axis_name="s")
    return pl.kernel(
        _sc_agg_body,
        out_type=jax.ShapeDtypeStruct((NC, N2, 128), jnp.float32),
        mesh=mesh,
        scratch_types=[
            pltpu.VMEM((EDGES_W,), jnp.int32),
            pltpu.VMEM((DHALF, CHUNK), jnp.int32),
            pltpu.VMEM((3, ZROWS, 128), jnp.float32),
            pltpu.VMEM_SHARED((N2, 128), jnp.float32),
            pltpu.SemaphoreType.DMA,
            pltpu.SemaphoreType.DMA,
        ],
    )


@functools.lru_cache(maxsize=None)
def _sc_deg():
    mesh = plsc.VectorSubcoreMesh(core_axis_name="c", subcore_axis_name="s")
    return pl.kernel(
        _sc_deg_body,
        out_type=jax.ShapeDtypeStruct((NC, N2), jnp.float32),
        mesh=mesh,
        compiler_params=pltpu.CompilerParams(needs_layout_passes=False),
        scratch_types=[
            pltpu.VMEM((ROWS_W, CHUNK), jnp.int32),
            pltpu.VMEM((N2,), jnp.float32),
            pltpu.VMEM((TILE_ROWS,), jnp.float32),
            pltpu.VMEM((TILE_ROWS,), jnp.float32),
            pltpu.VMEM_SHARED((NS, N2), jnp.float32),
            pltpu.SemaphoreType.DMA,
        ],
    )


_BM = 1000


def _layer_body(h_ref, a0_ref, a1_ref, d0_ref, d1_ref, ws_ref, wn_ref, b_ref,
                out_ref):
    deg = d0_ref[...] + d1_ref[...]
    mean = (a0_ref[...] + a1_ref[...]) / jnp.maximum(deg, 1.0)
    acc = jnp.dot(h_ref[...], ws_ref[...], preferred_element_type=jnp.float32)
    acc = acc + jnp.dot(mean, wn_ref[...], preferred_element_type=jnp.float32)
    out_ref[...] = jnp.maximum(acc + b_ref[...], 0.0)


def _layer(h, a0, a1, d0, d1, ws, wn, b):
    grid = N // _BM
    return pl.pallas_call(
        _layer_body,
        grid=(grid,),
        in_specs=[
            pl.BlockSpec((_BM, 128), lambda i: (i, 0)),
            pl.BlockSpec((_BM, 128), lambda i: (i, 0)),
            pl.BlockSpec((_BM, 128), lambda i: (i, 0)),
            pl.BlockSpec((_BM, 1), lambda i: (i, 0)),
            pl.BlockSpec((_BM, 1), lambda i: (i, 0)),
            pl.BlockSpec((128, 128), lambda i: (0, 0)),
            pl.BlockSpec((128, 128), lambda i: (0, 0)),
            pl.BlockSpec((1, 128), lambda i: (0, 0)),
        ],
        out_specs=pl.BlockSpec((_BM, 128), lambda i: (i, 0)),
        out_shape=jax.ShapeDtypeStruct((N, 128), jnp.float32),
    )(h, a0, a1, d0, d1, ws, wn, b.reshape(1, 128))


def _pool_body(b_ref, h1_ref, h2_ref, h3_ref, w1a_ref, w1b_ref, w1c_ref,
               fb1_ref, w2_ref, fb2_ref, out_ref, s1, s2, s3, cnt):
    i = pl.program_id(0)

    @pl.when(i == 0)
    def _():
        s1[...] = jnp.zeros_like(s1)
        s2[...] = jnp.zeros_like(s2)
        s3[...] = jnp.zeros_like(s3)
        cnt[...] = jnp.zeros_like(cnt)

    oh = (b_ref[...] == lax.broadcasted_iota(jnp.int32, (1, G), 1)
          ).astype(jnp.float32)
    dn = (((0,), (0,)), ((), ()))
    s1[...] += lax.dot_general(oh, h1_ref[...], dn,
                               preferred_element_type=jnp.float32)
    s2[...] += lax.dot_general(oh, h2_ref[...], dn,
                               preferred_element_type=jnp.float32)
    s3[...] += lax.dot_general(oh, h3_ref[...], dn,
                               preferred_element_type=jnp.float32)
    cnt[...] += lax.dot_general(oh, jnp.ones_like(h1_ref[...]), dn,
                                preferred_element_type=jnp.float32)

    @pl.when(i == pl.num_programs(0) - 1)
    def _():
        inv = 1.0 / jnp.maximum(cnt[...], 1.0)
        t = jnp.dot(s1[...] * inv, w1a_ref[...],
                    preferred_element_type=jnp.float32)
        t = t + jnp.dot(s2[...] * inv, w1b_ref[...],
                        preferred_element_type=jnp.float32)
        t = t + jnp.dot(s3[...] * inv, w1c_ref[...],
                        preferred_element_type=jnp.float32)
        t = jnp.maximum(t + fb1_ref[...], 0.0)
        out_ref[...] = jnp.dot(t, w2_ref[...],
                               preferred_element_type=jnp.float32) + fb2_ref[...]


def _pool_mlp(batch, h1, h2, h3, fc1_W, fc1_b, fc2_W, fc2_b):
    grid = N // _BM
    w1a, w1b, w1c = fc1_W[:128], fc1_W[128:256], fc1_W[256:]
    return pl.pallas_call(
        _pool_body,
        grid=(grid,),
        in_specs=[
            pl.BlockSpec((_BM, 1), lambda i: (i, 0)),
            pl.BlockSpec((_BM, 128), lambda i: (i, 0)),
            pl.BlockSpec((_BM, 128), lambda i: (i, 0)),
            pl.BlockSpec((_BM, 128), lambda i: (i, 0)),
            pl.BlockSpec((128, 128), lambda i: (0, 0)),
            pl.BlockSpec((128, 128), lambda i: (0, 0)),
            pl.BlockSpec((128, 128), lambda i: (0, 0)),
            pl.BlockSpec((1, 128), lambda i: (0, 0)),
            pl.BlockSpec((128, C), lambda i: (0, 0)),
            pl.BlockSpec((1, C), lambda i: (0, 0)),
        ],
        out_specs=pl.BlockSpec((G, C), lambda i: (0, 0)),
        out_shape=jax.ShapeDtypeStruct((G, C), jnp.float32),
        scratch_shapes=[
            pltpu.VMEM((G, 128), jnp.float32),
            pltpu.VMEM((G, 128), jnp.float32),
            pltpu.VMEM((G, 128), jnp.float32),
            pltpu.VMEM((G, 128), jnp.float32),
        ],
    )(batch.reshape(N, 1), h1, h2, h3, w1a, w1b, w1c,
      fc1_b.reshape(1, 128), fc2_W, fc2_b.reshape(1, C))


def kernel(x, edge_index, batch, W_self_0, W_neigh_0, b_0, W_self_1, W_neigh_1,
           b_1, W_self_2, W_neigh_2, b_2, fc1_W, fc1_b, fc2_W, fc2_b):
    ei = _min_adjust(edge_index)
    src = ei[0].reshape(NW, EDGES_W)
    dst = ei[1].reshape(NW, ROWS_W, CHUNK)

    deg = _sc_deg()(dst)
    d0 = deg[0, :N].reshape(N, 1)
    d1 = deg[1, :N].reshape(N, 1)
    agg = _sc_pass()(x, src, dst)
    h1 = _layer(x, agg[0, :N], agg[1, :N], d0, d1, W_self_0, W_neigh_0, b_0)
    agg = _sc_pass()(h1, src, dst)
    h2 = _layer(h1, agg[0, :N], agg[1, :N], d0, d1, W_self_1, W_neigh_1, b_1)
    agg = _sc_pass()(h2, src, dst)
    h3 = _layer(h2, agg[0, :N], agg[1, :N], d0, d1, W_self_2, W_neigh_2, b_2)

    return _pool_mlp(batch, h1, h2, h3, fc1_W, fc1_b, fc2_W, fc2_b)


# speedup vs baseline: 1.4299x; 1.0308x over previous
import functools

import jax
import jax.numpy as jnp
from jax import lax
from jax.experimental import pallas as pl
from jax.experimental.pallas import tpu as pltpu
from jax.experimental.pallas import tpu_sc as plsc

N = 10000
E = 320000
D = 128
H = 128
G = 64
C = 10

NC = 2
NS = 16
NW = NC * NS
CHUNK = 80
ROWS_TOT = E // CHUNK
ROWS_W = ROWS_TOT // NW
EDGES_W = E // NW
N2 = 10240
TILE_ROWS = N2 // NS
ZROWS = 80
DHALF = 64


def _minadj_body(ei_ref, out_ref):
    ei = ei_ref[...]
    out_ref[...] = ei - jnp.min(ei)


def _min_adjust(edge_index):
    return pl.pallas_call(
        _minadj_body,
        out_shape=jax.ShapeDtypeStruct((2, E), jnp.int32),
    )(edge_index)


def _fill16(ref, nrows, ncols, val):
    per_row = ncols // 16

    def body(i, _):
        r = i // per_row
        c = (i % per_row) * 16
        ref[r, pl.ds(c, 16)] = jnp.full((16,), val, jnp.float32)
        return 0

    lax.fori_loop(0, nrows * per_row, body, 0)


def _sc_agg_body(h_hbm, src_hbm, dst_hbm, agg_out, srcv, dstv, rows, agg_sp, sem, ssem):
    cid = lax.axis_index("c")
    sid = lax.axis_index("s")
    wid = sid * NC + cid

    _fill16(rows.at[0], ZROWS, 128, 0.0)
    for k in range(TILE_ROWS // ZROWS):
        pltpu.async_copy(rows.at[0],
                         agg_sp.at[pl.ds(sid * TILE_ROWS + k * ZROWS, ZROWS)],
                         ssem)
    for k in range(TILE_ROWS // ZROWS):
        pltpu.make_async_copy(rows.at[0], agg_sp.at[pl.ds(0, ZROWS)],
                              ssem).wait()
    plsc.subcore_barrier()

    pltpu.sync_copy(src_hbm.at[wid], srcv)
    pltpu.sync_copy(dst_hbm.at[wid, pl.ds(0, DHALF)], dstv)

    def gather_idx(j):
        return srcv.at[pl.ds(j * CHUNK, CHUNK)]

    def scat_wait():
        pltpu.make_async_copy(rows.at[0], agg_sp.at[pl.ds(0, ZROWS)],
                              ssem).wait()

    pltpu.async_copy(h_hbm.at[gather_idx(0)], rows.at[0], sem)
    pltpu.async_copy(h_hbm.at[gather_idx(1)], rows.at[1], sem)

    def chunk_body(j, _):
        b = lax.rem(j, 3)
        pltpu.make_async_copy(h_hbm.at[gather_idx(j)], rows.at[b], sem).wait()

        @pl.when(j >= 1)
        def _():
            scat_wait()

        @pl.when(j == DHALF)
        def _():
            pltpu.sync_copy(dst_hbm.at[wid, pl.ds(DHALF, ROWS_W - DHALF)],
                            dstv.at[pl.ds(0, ROWS_W - DHALF)])

        @pl.when(j + 2 < ROWS_W)
        def _():
            pltpu.async_copy(h_hbm.at[gather_idx(j + 2)],
                             rows.at[lax.rem(j + 2, 3)], sem)

        pltpu.async_copy(rows.at[b], agg_sp.at[dstv.at[lax.rem(j, DHALF)]],
                         ssem, add=True)
        return 0

    lax.fori_loop(0, ROWS_W, chunk_body, 0)
    scat_wait()
    plsc.subcore_barrier()

    r = sid * TILE_ROWS
    pltpu.sync_copy(agg_sp.at[pl.ds(r, TILE_ROWS)],
                    agg_out.at[cid, pl.ds(r, TILE_ROWS)])


def _sc_deg_body(dst_hbm, deg_out, dstv, degloc, tmpa, tmpb, parts_sp, sem):
    cid = lax.axis_index("c")
    sid = lax.axis_index("s")
    wid = sid * NC + cid

    def zero16(i, _):
        degloc[pl.ds(i * 16, 16)] = jnp.zeros((16,), jnp.float32)
        return 0

    lax.fori_loop(0, N2 // 16, zero16, 0)
    pltpu.sync_copy(dst_hbm.at[wid], dstv)

    def hist_row(j, _):
        for c in range(CHUNK // 16):
            idx = dstv[j, pl.ds(c * 16, 16)]
            cnt, last = plsc.scan_count(idx)
            plsc.addupdate_scatter(degloc, [idx], cnt.astype(jnp.float32),
                                   mask=last)
        return 0

    lax.fori_loop(0, ROWS_W, hist_row, 0)

    pltpu.sync_copy(degloc, parts_sp.at[sid])
    plsc.subcore_barrier()
    base = sid * TILE_ROWS
    pltpu.sync_copy(parts_sp.at[0, pl.ds(base, TILE_ROWS)], tmpa)
    for r in range(1, NS):
        pltpu.sync_copy(parts_sp.at[r, pl.ds(base, TILE_ROWS)], tmpb)
        for c in range(TILE_ROWS // 16):
            s = pl.ds(c * 16, 16)
            tmpa[s] = tmpa[s] + tmpb[s]
    pltpu.sync_copy(tmpa, deg_out.at[cid, pl.ds(base, TILE_ROWS)])


@functools.lru_cache(maxsize=None)
def _sc_pass():
    mesh = plsc.VectorSubcoreMesh(core_axis_name="c", subcore_axis_name="s")
    return pl.kernel(
        _sc_agg_body,
        out_type=jax.ShapeDtypeStruct((NC, N2, 128), jnp.float32),
        mesh=mesh,
        scratch_types=[
            pltpu.VMEM((EDGES_W,), jnp.int32),
            pltpu.VMEM((DHALF, CHUNK), jnp.int32),
            pltpu.VMEM((3, ZROWS, 128), jnp.float32),
            pltpu.VMEM_SHARED((N2, 128), jnp.float32),
            pltpu.SemaphoreType.DMA,
            pltpu.SemaphoreType.DMA,
        ],
    )


@functools.lru_cache(maxsize=None)
def _sc_deg():
    mesh = plsc.VectorSubcoreMesh(core_axis_name="c", subcore_axis_name="s")
    return pl.kernel(
        _sc_deg_body,
        out_type=jax.ShapeDtypeStruct((NC, N2), jnp.float32),
        mesh=mesh,
        compiler_params=pltpu.CompilerParams(needs_layout_passes=False),
        scratch_types=[
            pltpu.VMEM((ROWS_W, CHUNK), jnp.int32),
            pltpu.VMEM((N2,), jnp.float32),
            pltpu.VMEM((TILE_ROWS,), jnp.float32),
            pltpu.VMEM((TILE_ROWS,), jnp.float32),
            pltpu.VMEM_SHARED((NS, N2), jnp.float32),
            pltpu.SemaphoreType.DMA,
        ],
    )


_BM = 1000


def _layer_body(h_ref, a0_ref, a1_ref, d0_ref, d1_ref, ws_ref, wn_ref, b_ref,
                out_ref):
    deg = d0_ref[...] + d1_ref[...]
    mean = (a0_ref[...] + a1_ref[...]) / jnp.maximum(deg, 1.0)
    acc = jnp.dot(h_ref[...], ws_ref[...], preferred_element_type=jnp.float32)
    acc = acc + jnp.dot(mean, wn_ref[...], preferred_element_type=jnp.float32)
    out_ref[...] = jnp.maximum(acc + b_ref[...], 0.0)


def _layer(h, a0, a1, d0, d1, ws, wn, b):
    grid = N // _BM
    return pl.pallas_call(
        _layer_body,
        grid=(grid,),
        in_specs=[
            pl.BlockSpec((_BM, 128), lambda i: (i, 0)),
            pl.BlockSpec((_BM, 128), lambda i: (i, 0)),
            pl.BlockSpec((_BM, 128), lambda i: (i, 0)),
            pl.BlockSpec((_BM, 1), lambda i: (i, 0)),
            pl.BlockSpec((_BM, 1), lambda i: (i, 0)),
            pl.BlockSpec((128, 128), lambda i: (0, 0)),
            pl.BlockSpec((128, 128), lambda i: (0, 0)),
            pl.BlockSpec((1, 128), lambda i: (0, 0)),
        ],
        out_specs=pl.BlockSpec((_BM, 128), lambda i: (i, 0)),
        out_shape=jax.ShapeDtypeStruct((N, 128), jnp.float32),
    )(h, a0, a1, d0, d1, ws, wn, b.reshape(1, 128))


def _onehot(b_ref):
    return (b_ref[...] == lax.broadcasted_iota(jnp.int32, (1, G), 1)
            ).astype(jnp.float32)


_DN = (((0,), (0,)), ((), ()))


def _layer2_body(h_ref, a0_ref, a1_ref, d0_ref, d1_ref, ws_ref, wn_ref, b_ref,
                 bat_ref, out_ref, p1_ref, cnt_ref, sp1, scnt):
    i = pl.program_id(0)

    @pl.when(i == 0)
    def _():
        sp1[...] = jnp.zeros_like(sp1)
        scnt[...] = jnp.zeros_like(scnt)

    deg = d0_ref[...] + d1_ref[...]
    mean = (a0_ref[...] + a1_ref[...]) / jnp.maximum(deg, 1.0)
    acc = jnp.dot(h_ref[...], ws_ref[...], preferred_element_type=jnp.float32)
    acc = acc + jnp.dot(mean, wn_ref[...], preferred_element_type=jnp.float32)
    out_ref[...] = jnp.maximum(acc + b_ref[...], 0.0)

    oh = _onehot(bat_ref)
    sp1[...] += lax.dot_general(oh, h_ref[...], _DN,
                                preferred_element_type=jnp.float32)
    scnt[...] += lax.dot_general(oh, jnp.ones_like(h_ref[...]), _DN,
                                 preferred_element_type=jnp.float32)

    @pl.when(i == pl.num_programs(0) - 1)
    def _():
        p1_ref[...] = sp1[...]
        cnt_ref[...] = scnt[...]


def _layer2(h, a0, a1, d0, d1, ws, wn, b, batch):
    grid = N // _BM
    blk = pl.BlockSpec((_BM, 128), lambda i: (i, 0))
    w128 = pl.BlockSpec((128, 128), lambda i: (0, 0))
    g128 = pl.BlockSpec((G, 128), lambda i: (0, 0))
    return pl.pallas_call(
        _layer2_body,
        grid=(grid,),
        in_specs=[
            blk, blk, blk,
            pl.BlockSpec((_BM, 1), lambda i: (i, 0)),
            pl.BlockSpec((_BM, 1), lambda i: (i, 0)),
            w128, w128,
            pl.BlockSpec((1, 128), lambda i: (0, 0)),
            pl.BlockSpec((_BM, 1), lambda i: (i, 0)),
        ],
        out_specs=[blk, g128, g128],
        out_shape=[
            jax.ShapeDtypeStruct((N, 128), jnp.float32),
            jax.ShapeDtypeStruct((G, 128), jnp.float32),
            jax.ShapeDtypeStruct((G, 128), jnp.float32),
        ],
        scratch_shapes=[
            pltpu.VMEM((G, 128), jnp.float32),
            pltpu.VMEM((G, 128), jnp.float32),
        ],
    )(h, a0, a1, d0, d1, ws, wn, b.reshape(1, 128), batch.reshape(N, 1))


def _layer3_body(h_ref, a0_ref, a1_ref, d0_ref, d1_ref, ws_ref, wn_ref, b_ref,
                 bat_ref, p1_ref, cnt_ref, w1a_ref, w1b_ref, w1c_ref, fb1_ref,
                 w2_ref, fb2_ref, out_ref, sp2, sp3):
    i = pl.program_id(0)

    @pl.when(i == 0)
    def _():
        sp2[...] = jnp.zeros_like(sp2)
        sp3[...] = jnp.zeros_like(sp3)

    deg = d0_ref[...] + d1_ref[...]
    mean = (a0_ref[...] + a1_ref[...]) / jnp.maximum(deg, 1.0)
    acc = jnp.dot(h_ref[...], ws_ref[...], preferred_element_type=jnp.float32)
    acc = acc + jnp.dot(mean, wn_ref[...], preferred_element_type=jnp.float32)
    h3 = jnp.maximum(acc + b_ref[...], 0.0)

    oh = _onehot(bat_ref)
    sp2[...] += lax.dot_general(oh, h_ref[...], _DN,
                                preferred_element_type=jnp.float32)
    sp3[...] += lax.dot_general(oh, h3, _DN,
                                preferred_element_type=jnp.float32)

    @pl.when(i == pl.num_programs(0) - 1)
    def _():
        inv = 1.0 / jnp.maximum(cnt_ref[...], 1.0)
        t = jnp.dot(p1_ref[...] * inv, w1a_ref[...],
                    preferred_element_type=jnp.float32)
        t = t + jnp.dot(sp2[...] * inv, w1b_ref[...],
                        preferred_element_type=jnp.float32)
        t = t + jnp.dot(sp3[...] * inv, w1c_ref[...],
                        preferred_element_type=jnp.float32)
        t = jnp.maximum(t + fb1_ref[...], 0.0)
        out_ref[...] = jnp.dot(t, w2_ref[...],
                               preferred_element_type=jnp.float32) + fb2_ref[...]


def _layer3(h, a0, a1, d0, d1, ws, wn, b, batch, p1, cnt,
            fc1_W, fc1_b, fc2_W, fc2_b):
    grid = N // _BM
    blk = pl.BlockSpec((_BM, 128), lambda i: (i, 0))
    w128 = pl.BlockSpec((128, 128), lambda i: (0, 0))
    g128 = pl.BlockSpec((G, 128), lambda i: (0, 0))
    w1a, w1b, w1c = fc1_W[:128], fc1_W[128:256], fc1_W[256:]
    return pl.pallas_call(
        _layer3_body,
        grid=(grid,),
        in_specs=[
            blk, blk, blk,
            pl.BlockSpec((_BM, 1), lambda i: (i, 0)),
            pl.BlockSpec((_BM, 1), lambda i: (i, 0)),
            w128, w128,
            pl.BlockSpec((1, 128), lambda i: (0, 0)),
            pl.BlockSpec((_BM, 1), lambda i: (i, 0)),
            g128, g128,
            w128, w128, w128,
            pl.BlockSpec((1, 128), lambda i: (0, 0)),
            pl.BlockSpec((128, C), lambda i: (0, 0)),
            pl.BlockSpec((1, C), lambda i: (0, 0)),
        ],
        out_specs=pl.BlockSpec((G, C), lambda i: (0, 0)),
        out_shape=jax.ShapeDtypeStruct((G, C), jnp.float32),
        scratch_shapes=[
            pltpu.VMEM((G, 128), jnp.float32),
            pltpu.VMEM((G, 128), jnp.float32),
        ],
    )(h, a0, a1, d0, d1, ws, wn, b.reshape(1, 128), batch.reshape(N, 1),
      p1, cnt, w1a, w1b, w1c, fc1_b.reshape(1, 128), fc2_W,
      fc2_b.reshape(1, C))


def kernel(x, edge_index, batch, W_self_0, W_neigh_0, b_0, W_self_1, W_neigh_1,
           b_1, W_self_2, W_neigh_2, b_2, fc1_W, fc1_b, fc2_W, fc2_b):
    ei = _min_adjust(edge_index)
    src = ei[0].reshape(NW, EDGES_W)
    dst = ei[1].reshape(NW, ROWS_W, CHUNK)

    deg = _sc_deg()(dst)
    d0 = deg[0, :N].reshape(N, 1)
    d1 = deg[1, :N].reshape(N, 1)
    agg = _sc_pass()(x, src, dst)
    h1 = _layer(x, agg[0, :N], agg[1, :N], d0, d1, W_self_0, W_neigh_0, b_0)
    agg = _sc_pass()(h1, src, dst)
    h2, p1, cnt = _layer2(h1, agg[0, :N], agg[1, :N], d0, d1,
                          W_self_1, W_neigh_1, b_1, batch)
    agg = _sc_pass()(h2, src, dst)
    return _layer3(h2, agg[0, :N], agg[1, :N], d0, d1, W_self_2, W_neigh_2,
                   b_2, batch, p1, cnt, fc1_W, fc1_b, fc2_W, fc2_b)


# speedup vs baseline: 1.4314x; 1.0010x over previous
import functools

import jax
import jax.numpy as jnp
from jax import lax
from jax.experimental import pallas as pl
from jax.experimental.pallas import tpu as pltpu
from jax.experimental.pallas import tpu_sc as plsc

N = 10000
E = 320000
D = 128
H = 128
G = 64
C = 10

NC = 2
NS = 16
NW = NC * NS
CHUNK = 80
ROWS_TOT = E // CHUNK
ROWS_W = ROWS_TOT // NW
EDGES_W = E // NW
N2 = 10240
TILE_ROWS = N2 // NS
ZROWS = 80
DHALF = 64


def _minadj_body(ei_ref, out_ref):
    ei = ei_ref[...]
    out_ref[...] = ei - jnp.min(ei)


def _min_adjust(edge_index):
    return pl.pallas_call(
        _minadj_body,
        out_shape=jax.ShapeDtypeStruct((2, E), jnp.int32),
    )(edge_index)


def _fill16(ref, nrows, ncols, val):
    per_row = ncols // 16

    def body(i, _):
        r = i // per_row
        c = (i % per_row) * 16
        ref[r, pl.ds(c, 16)] = jnp.full((16,), val, jnp.float32)
        return 0

    lax.fori_loop(0, nrows * per_row, body, 0)


def _sc_agg_body(h_hbm, src_hbm, dst_hbm, agg_out, srcv, dstv, rows, agg_sp, sem, ssem):
    cid = lax.axis_index("c")
    sid = lax.axis_index("s")
    wid = sid * NC + cid

    _fill16(rows.at[0], ZROWS, 128, 0.0)
    for k in range(TILE_ROWS // ZROWS):
        pltpu.async_copy(rows.at[0],
                         agg_sp.at[pl.ds(sid * TILE_ROWS + k * ZROWS, ZROWS)],
                         ssem)
    for k in range(TILE_ROWS // ZROWS):
        pltpu.make_async_copy(rows.at[0], agg_sp.at[pl.ds(0, ZROWS)],
                              ssem).wait()
    plsc.subcore_barrier()

    pltpu.sync_copy(src_hbm.at[wid], srcv)
    pltpu.sync_copy(dst_hbm.at[wid, pl.ds(0, DHALF)], dstv)

    def gather_idx(j):
        return srcv.at[pl.ds(j * CHUNK, CHUNK)]

    def scat_wait():
        pltpu.make_async_copy(rows.at[0], agg_sp.at[pl.ds(0, ZROWS)],
                              ssem).wait()

    pltpu.async_copy(h_hbm.at[gather_idx(0)], rows.at[0], sem)
    pltpu.async_copy(h_hbm.at[gather_idx(1)], rows.at[1], sem)

    def do_chunk(j, m, issue):
        pltpu.make_async_copy(h_hbm.at[gather_idx(j)], rows.at[m], sem).wait()
        if m > 0 or isinstance(j, int):
            scat_wait()
        else:
            @pl.when(j >= 1)
            def _():
                scat_wait()
        if m == 1 and not isinstance(j, int):
            @pl.when(j == DHALF)
            def _():
                pltpu.sync_copy(dst_hbm.at[wid, pl.ds(DHALF, ROWS_W - DHALF)],
                                dstv.at[pl.ds(0, ROWS_W - DHALF)])
        if issue:
            pltpu.async_copy(h_hbm.at[gather_idx(j + 2)],
                             rows.at[(m + 2) % 3], sem)
        pltpu.async_copy(rows.at[m], agg_sp.at[dstv.at[lax.rem(j, DHALF)]],
                         ssem, add=True)

    def chunk_body(k, _):
        for m in range(3):
            do_chunk(k * 3 + m, m, True)
        return 0

    lax.fori_loop(0, (ROWS_W - 2) // 3, chunk_body, 0)
    do_chunk(ROWS_W - 2, 0, False)
    do_chunk(ROWS_W - 1, 1, False)
    scat_wait()
    plsc.subcore_barrier()

    r = sid * TILE_ROWS
    pltpu.sync_copy(agg_sp.at[pl.ds(r, TILE_ROWS)],
                    agg_out.at[cid, pl.ds(r, TILE_ROWS)])


def _sc_deg_body(dst_hbm, deg_out, dstv, degloc, tmpa, tmpb, parts_sp, sem):
    cid = lax.axis_index("c")
    sid = lax.axis_index("s")
    wid = sid * NC + cid

    def zero16(i, _):
        degloc[pl.ds(i * 16, 16)] = jnp.zeros((16,), jnp.float32)
        return 0

    lax.fori_loop(0, N2 // 16, zero16, 0)
    pltpu.sync_copy(dst_hbm.at[wid], dstv)

    def hist_row(j, _):
        for c in range(CHUNK // 16):
            idx = dstv[j, pl.ds(c * 16, 16)]
            cnt, last = plsc.scan_count(idx)
            plsc.addupdate_scatter(degloc, [idx], cnt.astype(jnp.float32),
                                   mask=last)
        return 0

    lax.fori_loop(0, ROWS_W, hist_row, 0)

    pltpu.sync_copy(degloc, parts_sp.at[sid])
    plsc.subcore_barrier()
    base = sid * TILE_ROWS
    pltpu.sync_copy(parts_sp.at[0, pl.ds(base, TILE_ROWS)], tmpa)
    for r in range(1, NS):
        pltpu.sync_copy(parts_sp.at[r, pl.ds(base, TILE_ROWS)], tmpb)
        for c in range(TILE_ROWS // 16):
            s = pl.ds(c * 16, 16)
            tmpa[s] = tmpa[s] + tmpb[s]
    pltpu.sync_copy(tmpa, deg_out.at[cid, pl.ds(base, TILE_ROWS)])


@functools.lru_cache(maxsize=None)
def _sc_pass():
    mesh = plsc.VectorSubcoreMesh(core_axis_name="c", subcore_axis_name="s")
    return pl.kernel(
        _sc_agg_body,
        out_type=jax.ShapeDtypeStruct((NC, N2, 128), jnp.float32),
        mesh=mesh,
        scratch_types=[
            pltpu.VMEM((EDGES_W,), jnp.int32),
            pltpu.VMEM((DHALF, CHUNK), jnp.int32),
            pltpu.VMEM((3, ZROWS, 128), jnp.float32),
            pltpu.VMEM_SHARED((N2, 128), jnp.float32),
            pltpu.SemaphoreType.DMA,
            pltpu.SemaphoreType.DMA,
        ],
    )


@functools.lru_cache(maxsize=None)
def _sc_deg():
    mesh = plsc.VectorSubcoreMesh(core_axis_name="c", subcore_axis_name="s")
    return pl.kernel(
        _sc_deg_body,
        out_type=jax.ShapeDtypeStruct((NC, N2), jnp.float32),
        mesh=mesh,
        compiler_params=pltpu.CompilerParams(needs_layout_passes=False),
        scratch_types=[
            pltpu.VMEM((ROWS_W, CHUNK), jnp.int32),
            pltpu.VMEM((N2,), jnp.float32),
            pltpu.VMEM((TILE_ROWS,), jnp.float32),
            pltpu.VMEM((TILE_ROWS,), jnp.float32),
            pltpu.VMEM_SHARED((NS, N2), jnp.float32),
            pltpu.SemaphoreType.DMA,
        ],
    )


_BM = 1000


def _layer_body(h_ref, a0_ref, a1_ref, d0_ref, d1_ref, ws_ref, wn_ref, b_ref,
                out_ref):
    deg = d0_ref[...] + d1_ref[...]
    mean = (a0_ref[...] + a1_ref[...]) / jnp.maximum(deg, 1.0)
    acc = jnp.dot(h_ref[...], ws_ref[...], preferred_element_type=jnp.float32)
    acc = acc + jnp.dot(mean, wn_ref[...], preferred_element_type=jnp.float32)
    out_ref[...] = jnp.maximum(acc + b_ref[...], 0.0)


def _layer(h, a0, a1, d0, d1, ws, wn, b):
    grid = N // _BM
    return pl.pallas_call(
        _layer_body,
        grid=(grid,),
        in_specs=[
            pl.BlockSpec((_BM, 128), lambda i: (i, 0)),
            pl.BlockSpec((_BM, 128), lambda i: (i, 0)),
            pl.BlockSpec((_BM, 128), lambda i: (i, 0)),
            pl.BlockSpec((_BM, 1), lambda i: (i, 0)),
            pl.BlockSpec((_BM, 1), lambda i: (i, 0)),
            pl.BlockSpec((128, 128), lambda i: (0, 0)),
            pl.BlockSpec((128, 128), lambda i: (0, 0)),
            pl.BlockSpec((1, 128), lambda i: (0, 0)),
        ],
        out_specs=pl.BlockSpec((_BM, 128), lambda i: (i, 0)),
        out_shape=jax.ShapeDtypeStruct((N, 128), jnp.float32),
    )(h, a0, a1, d0, d1, ws, wn, b.reshape(1, 128))


def _onehot(b_ref):
    return (b_ref[...] == lax.broadcasted_iota(jnp.int32, (1, G), 1)
            ).astype(jnp.float32)


_DN = (((0,), (0,)), ((), ()))


def _layer2_body(h_ref, a0_ref, a1_ref, d0_ref, d1_ref, ws_ref, wn_ref, b_ref,
                 bat_ref, out_ref, p1_ref, cnt_ref, sp1, scnt):
    i = pl.program_id(0)

    @pl.when(i == 0)
    def _():
        sp1[...] = jnp.zeros_like(sp1)
        scnt[...] = jnp.zeros_like(scnt)

    deg = d0_ref[...] + d1_ref[...]
    mean = (a0_ref[...] + a1_ref[...]) / jnp.maximum(deg, 1.0)
    acc = jnp.dot(h_ref[...], ws_ref[...], preferred_element_type=jnp.float32)
    acc = acc + jnp.dot(mean, wn_ref[...], preferred_element_type=jnp.float32)
    out_ref[...] = jnp.maximum(acc + b_ref[...], 0.0)

    oh = _onehot(bat_ref)
    sp1[...] += lax.dot_general(oh, h_ref[...], _DN,
                                preferred_element_type=jnp.float32)
    scnt[...] += lax.dot_general(oh, jnp.ones_like(h_ref[...]), _DN,
                                 preferred_element_type=jnp.float32)

    @pl.when(i == pl.num_programs(0) - 1)
    def _():
        p1_ref[...] = sp1[...]
        cnt_ref[...] = scnt[...]


def _layer2(h, a0, a1, d0, d1, ws, wn, b, batch):
    grid = N // _BM
    blk = pl.BlockSpec((_BM, 128), lambda i: (i, 0))
    w128 = pl.BlockSpec((128, 128), lambda i: (0, 0))
    g128 = pl.BlockSpec((G, 128), lambda i: (0, 0))
    return pl.pallas_call(
        _layer2_body,
        grid=(grid,),
        in_specs=[
            blk, blk, blk,
            pl.BlockSpec((_BM, 1), lambda i: (i, 0)),
            pl.BlockSpec((_BM, 1), lambda i: (i, 0)),
            w128, w128,
            pl.BlockSpec((1, 128), lambda i: (0, 0)),
            pl.BlockSpec((_BM, 1), lambda i: (i, 0)),
        ],
        out_specs=[blk, g128, g128],
        out_shape=[
            jax.ShapeDtypeStruct((N, 128), jnp.float32),
            jax.ShapeDtypeStruct((G, 128), jnp.float32),
            jax.ShapeDtypeStruct((G, 128), jnp.float32),
        ],
        scratch_shapes=[
            pltpu.VMEM((G, 128), jnp.float32),
            pltpu.VMEM((G, 128), jnp.float32),
        ],
    )(h, a0, a1, d0, d1, ws, wn, b.reshape(1, 128), batch.reshape(N, 1))


def _layer3_body(h_ref, a0_ref, a1_ref, d0_ref, d1_ref, ws_ref, wn_ref, b_ref,
                 bat_ref, p1_ref, cnt_ref, w1a_ref, w1b_ref, w1c_ref, fb1_ref,
                 w2_ref, fb2_ref, out_ref, sp2, sp3):
    i = pl.program_id(0)

    @pl.when(i == 0)
    def _():
        sp2[...] = jnp.zeros_like(sp2)
        sp3[...] = jnp.zeros_like(sp3)

    deg = d0_ref[...] + d1_ref[...]
    mean = (a0_ref[...] + a1_ref[...]) / jnp.maximum(deg, 1.0)
    acc = jnp.dot(h_ref[...], ws_ref[...], preferred_element_type=jnp.float32)
    acc = acc + jnp.dot(mean, wn_ref[...], preferred_element_type=jnp.float32)
    h3 = jnp.maximum(acc + b_ref[...], 0.0)

    oh = _onehot(bat_ref)
    sp2[...] += lax.dot_general(oh, h_ref[...], _DN,
                                preferred_element_type=jnp.float32)
    sp3[...] += lax.dot_general(oh, h3, _DN,
                                preferred_element_type=jnp.float32)

    @pl.when(i == pl.num_programs(0) - 1)
    def _():
        inv = 1.0 / jnp.maximum(cnt_ref[...], 1.0)
        t = jnp.dot(p1_ref[...] * inv, w1a_ref[...],
                    preferred_element_type=jnp.float32)
        t = t + jnp.dot(sp2[...] * inv, w1b_ref[...],
                        preferred_element_type=jnp.float32)
        t = t + jnp.dot(sp3[...] * inv, w1c_ref[...],
                        preferred_element_type=jnp.float32)
        t = jnp.maximum(t + fb1_ref[...], 0.0)
        out_ref[...] = jnp.dot(t, w2_ref[...],
                               preferred_element_type=jnp.float32) + fb2_ref[...]


def _layer3(h, a0, a1, d0, d1, ws, wn, b, batch, p1, cnt,
            fc1_W, fc1_b, fc2_W, fc2_b):
    grid = N // _BM
    blk = pl.BlockSpec((_BM, 128), lambda i: (i, 0))
    w128 = pl.BlockSpec((128, 128), lambda i: (0, 0))
    g128 = pl.BlockSpec((G, 128), lambda i: (0, 0))
    w1a, w1b, w1c = fc1_W[:128], fc1_W[128:256], fc1_W[256:]
    return pl.pallas_call(
        _layer3_body,
        grid=(grid,),
        in_specs=[
            blk, blk, blk,
            pl.BlockSpec((_BM, 1), lambda i: (i, 0)),
            pl.BlockSpec((_BM, 1), lambda i: (i, 0)),
            w128, w128,
            pl.BlockSpec((1, 128), lambda i: (0, 0)),
            pl.BlockSpec((_BM, 1), lambda i: (i, 0)),
            g128, g128,
            w128, w128, w128,
            pl.BlockSpec((1, 128), lambda i: (0, 0)),
            pl.BlockSpec((128, C), lambda i: (0, 0)),
            pl.BlockSpec((1, C), lambda i: (0, 0)),
        ],
        out_specs=pl.BlockSpec((G, C), lambda i: (0, 0)),
        out_shape=jax.ShapeDtypeStruct((G, C), jnp.float32),
        scratch_shapes=[
            pltpu.VMEM((G, 128), jnp.float32),
            pltpu.VMEM((G, 128), jnp.float32),
        ],
    )(h, a0, a1, d0, d1, ws, wn, b.reshape(1, 128), batch.reshape(N, 1),
      p1, cnt, w1a, w1b, w1c, fc1_b.reshape(1, 128), fc2_W,
      fc2_b.reshape(1, C))


def kernel(x, edge_index, batch, W_self_0, W_neigh_0, b_0, W_self_1, W_neigh_1,
           b_1, W_self_2, W_neigh_2, b_2, fc1_W, fc1_b, fc2_W, fc2_b):
    ei = _min_adjust(edge_index)
    src = ei[0].reshape(NW, EDGES_W)
    dst = ei[1].reshape(NW, ROWS_W, CHUNK)

    deg = _sc_deg()(dst)
    d0 = deg[0, :N].reshape(N, 1)
    d1 = deg[1, :N].reshape(N, 1)
    agg = _sc_pass()(x, src, dst)
    h1 = _layer(x, agg[0, :N], agg[1, :N], d0, d1, W_self_0, W_neigh_0, b_0)
    agg = _sc_pass()(h1, src, dst)
    h2, p1, cnt = _layer2(h1, agg[0, :N], agg[1, :N], d0, d1,
                          W_self_1, W_neigh_1, b_1, batch)
    agg = _sc_pass()(h2, src, dst)
    return _layer3(h2, agg[0, :N], agg[1, :N], d0, d1, W_self_2, W_neigh_2,
                   b_2, batch, p1, cnt, fc1_W, fc1_b, fc2_W, fc2_b)
